# baseline (speedup 1.0000x reference)
import jax
import jax.numpy as jnp
from jax.experimental import pallas as pl
from jax.experimental.pallas import tpu as pltpu

_E = 64
_TOPK = 8
_D = 1024
_I = 512
_IS = _I
_T = 64


def _moe_kernel(x_ref, gate_w_ref, wgu_ref, bgu_ref, wd_ref,
                wsgu_ref, bs_ref, wsd_ref,
                out_ref,
                xt_ref, wts_ref, acc_ref):
    e = pl.program_id(0)

    @pl.when(e == 0)
    def _prologue():
        xt = x_ref[...].T
        xt_ref[...] = xt.astype(jnp.bfloat16)
        logits = jax.lax.dot(gate_w_ref[...], xt,
                             preferred_element_type=jnp.float32)
        m = jnp.max(logits, axis=0, keepdims=True)
        p = jnp.exp(logits - m)
        p = p / jnp.sum(p, axis=0, keepdims=True)
        iota = jax.lax.broadcasted_iota(jnp.int32, p.shape, 0)
        work = p
        mask = jnp.zeros(p.shape, jnp.float32)
        for _ in range(_TOPK):
            mx = jnp.max(work, axis=0, keepdims=True)
            eq = work == mx
            first = jnp.min(jnp.where(eq, iota, _E), axis=0, keepdims=True)
            pick = iota == first
            mask = mask + pick.astype(jnp.float32)
            work = jnp.where(pick, -1.0, work)
        sel = p * mask
        wts_ref[...] = sel / jnp.sum(sel, axis=0, keepdims=True)
        xtb = xt.astype(jnp.bfloat16)
        gus = jax.lax.dot(wsgu_ref[...].astype(jnp.bfloat16), xtb,
                          preferred_element_type=jnp.float32) + bs_ref[...]
        g, u = gus[:_IS], gus[_IS:]
        acts = (g * jax.nn.sigmoid(g) * u).astype(jnp.bfloat16)
        acc_ref[...] = jax.lax.dot(wsd_ref[...].astype(jnp.bfloat16), acts,
                                   preferred_element_type=jnp.float32)

    xtb = xt_ref[...]
    gu = jax.lax.dot(wgu_ref[0].astype(jnp.bfloat16), xtb,
                     preferred_element_type=jnp.float32) + bgu_ref[0]
    g, u = gu[:_I], gu[_I:]
    act = (g * jax.nn.sigmoid(g) * u).astype(jnp.bfloat16)
    oe = jax.lax.dot(wd_ref[0].astype(jnp.bfloat16), act,
                     preferred_element_type=jnp.float32)
    w_row = wts_ref[pl.ds(e, 1), :]
    acc_ref[...] += oe * w_row

    @pl.when(e == _E - 1)
    def _epilogue():
        out_ref[...] = acc_ref[...].T


def kernel(hidden_states, gate_w, w_gate_up, b_gate_up, w_down,
           ws_gate_up, bs_gate_up, ws_down):
    bgu = b_gate_up.reshape(_E, 2 * _I, 1)
    bs = bs_gate_up.reshape(2 * _IS, 1)
    grid = (_E,)
    out = pl.pallas_call(
        _moe_kernel,
        grid=grid,
        in_specs=[
            pl.BlockSpec((_T, _D), lambda e: (0, 0)),
            pl.BlockSpec((_E, _D), lambda e: (0, 0)),
            pl.BlockSpec((1, 2 * _I, _D), lambda e: (e, 0, 0)),
            pl.BlockSpec((1, 2 * _I, 1), lambda e: (e, 0, 0)),
            pl.BlockSpec((1, _D, _I), lambda e: (e, 0, 0)),
            pl.BlockSpec((2 * _IS, _D), lambda e: (0, 0)),
            pl.BlockSpec((2 * _IS, 1), lambda e: (0, 0)),
            pl.BlockSpec((_D, _IS), lambda e: (0, 0)),
        ],
        out_specs=pl.BlockSpec((_T, _D), lambda e: (0, 0)),
        out_shape=jax.ShapeDtypeStruct((_T, _D), jnp.float32),
        scratch_shapes=[
            pltpu.VMEM((_D, _T), jnp.bfloat16),
            pltpu.VMEM((_E, _T), jnp.float32),
            pltpu.VMEM((_D, _T), jnp.float32),
        ],
        compiler_params=pltpu.CompilerParams(
            dimension_semantics=("arbitrary",),
        ),
    )(hidden_states, gate_w, w_gate_up, bgu, w_down, ws_gate_up, bs, ws_down)
    return out
